# baseline (speedup 1.0000x reference)
import functools
import jax
import jax.numpy as jnp
from jax import lax
from jax.experimental import pallas as pl
from jax.experimental.pallas import tpu as pltpu, tpu_sc as plsc

SEQ = 2048
B = 4
D = 1024
L = 16
NW = 32
CHUNK = SEQ // NW
SUB = 4
T = CHUNK // SUB


def _sc_body(x_hbm, pe_hbm, ev_hbm, period_hbm, wf_hbm, bf_hbm, et_hbm,
             out_hbm, x_v, o_v, pe_v, comb_v, per_v, wf_v, bf_v, et_v, ev_v,
             xs0, xs1, xs2, xs3, ps0, ps1, ps2, ps3, os0, os1, evs,
             s_per, s_wf, s_bf, s_et):
    wid = lax.axis_index("s") * 2 + lax.axis_index("c")
    base = wid * CHUNK

    x_sems = (xs0, xs1, xs2, xs3)
    p_sems = (ps0, ps1, ps2, ps3)
    o_sems = (os0, os1)

    def start_in(t, k):
        s0 = base + t * SUB
        pltpu.make_async_copy(x_hbm.at[pl.ds(s0, SUB)], x_v.at[k], x_sems[k]).start()
        pltpu.make_async_copy(pe_hbm.at[pl.ds(s0, SUB)], pe_v.at[k], p_sems[k]).start()

    def wait_in(t, k):
        s0 = base + t * SUB
        pltpu.make_async_copy(x_hbm.at[pl.ds(s0, SUB)], x_v.at[k], x_sems[k]).wait()
        pltpu.make_async_copy(pe_hbm.at[pl.ds(s0, SUB)], pe_v.at[k], p_sems[k]).wait()

    def start_out(t, k):
        s0 = base + t * SUB
        pltpu.make_async_copy(o_v.at[k], out_hbm.at[pl.ds(s0, SUB)], o_sems[k]).start()

    def wait_out(t, k):
        s0 = base + t * SUB
        pltpu.make_async_copy(o_v.at[k], out_hbm.at[pl.ds(s0, SUB)], o_sems[k]).wait()

    start_in(0, 0)
    start_in(1, 1)
    pltpu.make_async_copy(ev_hbm.at[pl.ds(base * B, CHUNK * B)], ev_v, evs).start()
    pltpu.make_async_copy(period_hbm, per_v.at[0, pl.ds(0, B)], s_per).start()
    pltpu.make_async_copy(wf_hbm, wf_v, s_wf).start()
    pltpu.make_async_copy(bf_hbm, bf_v, s_bf).start()
    pltpu.make_async_copy(et_hbm, et_v, s_et).start()
    pltpu.make_async_copy(period_hbm, per_v.at[0, pl.ds(0, B)], s_per).wait()
    pltpu.make_async_copy(wf_hbm, wf_v, s_wf).wait()
    pltpu.make_async_copy(bf_hbm, bf_v, s_bf).wait()
    pltpu.make_async_copy(et_hbm, et_v, s_et).wait()
    pltpu.make_async_copy(ev_hbm.at[pl.ds(base * B, CHUNK * B)], ev_v, evs).wait()

    for e in range(2):
        for b in range(B):
            p = jnp.take(per_v[0, :], jnp.full((L,), b, jnp.int32))
            def mk(j, _, e=e, b=b, p=p):
                sl = pl.ds(j * L, L)
                comb_v[e * B + b, sl] = (p * wf_v[0, sl] + bf_v[0, sl]
                                         + et_v[e, sl])
                return 0
            lax.fori_loop(0, D // L, mk, 0, unroll=4)

    def quad(tq, _):
        for phase in range(4):
            t = 4 * tq + phase
            k = phase
            ko = phase % 2
            @pl.when(t + 2 < T)
            def _():
                start_in(t + 2, (phase + 2) % 4)
            wait_in(t, k)
            @pl.when(t >= 2)
            def _():
                wait_out(t - 2, ko)
            vec = ev_v[pl.ds(t * SUB * B, L)]
            rs = [vec[lane] * B + (lane % B) for lane in range(L)]
            @plsc.parallel_loop(0, D // L, unroll=4)
            def dslice(j, rs=rs, k=k, ko=ko):
                sl = pl.ds(j * L, L)
                pes = [pe_v[k, s, sl] for s in range(SUB)]
                for lane in range(L):
                    s = lane // B
                    b = lane % B
                    o_v[ko, s, b, sl] = (x_v[k, s, b, sl] + pes[s]
                                         + comb_v[rs[lane], sl])
            start_out(t, ko)
        return 0

    lax.fori_loop(0, T // 4, quad, 0)
    wait_out(T - 2, 0)
    wait_out(T - 1, 1)


def kernel(x, period, events, W_freq, b_freq, event_table, pe):
    seq_len, batch, d = x.shape
    ev_flat = events.astype(jnp.int32).reshape(seq_len * batch)
    wf2 = W_freq.reshape(1, d)
    bf2 = b_freq.reshape(1, d)

    mesh = plsc.VectorSubcoreMesh(core_axis_name="c", subcore_axis_name="s")
    k = functools.partial(
        pl.kernel,
        mesh=mesh,
        out_type=jax.ShapeDtypeStruct((seq_len, batch, d), jnp.float32),
        scratch_types=[
            pltpu.VMEM((4, SUB, batch, d), jnp.float32),
            pltpu.VMEM((2, SUB, batch, d), jnp.float32),
            pltpu.VMEM((4, SUB, d), jnp.float32),
            pltpu.VMEM((2 * batch, d), jnp.float32),
            pltpu.VMEM((1, L), jnp.float32),
            pltpu.VMEM((1, d), jnp.float32),
            pltpu.VMEM((1, d), jnp.float32),
            pltpu.VMEM((2, d), jnp.float32),
            pltpu.VMEM((CHUNK * B,), jnp.int32),
            pltpu.SemaphoreType.DMA,
            pltpu.SemaphoreType.DMA,
            pltpu.SemaphoreType.DMA,
            pltpu.SemaphoreType.DMA,
            pltpu.SemaphoreType.DMA,
            pltpu.SemaphoreType.DMA,
            pltpu.SemaphoreType.DMA,
            pltpu.SemaphoreType.DMA,
            pltpu.SemaphoreType.DMA,
            pltpu.SemaphoreType.DMA,
            pltpu.SemaphoreType.DMA,
            pltpu.SemaphoreType.DMA,
            pltpu.SemaphoreType.DMA,
            pltpu.SemaphoreType.DMA,
            pltpu.SemaphoreType.DMA,
        ],
    )(_sc_body)
    return k(x, pe, ev_flat, period, wf2, bf2, event_table)


# speedup vs baseline: 1.0588x; 1.0588x over previous
import functools
import jax
import jax.numpy as jnp
from jax import lax
from jax.experimental import pallas as pl
from jax.experimental.pallas import tpu as pltpu, tpu_sc as plsc

SEQ = 2048
B = 4
D = 1024
L = 16
NW = 32
CHUNK = SEQ // NW
SUB = 4
T = CHUNK // SUB


def _sc_body(x_hbm, pe_hbm, ev_hbm, period_hbm, wf_hbm, bf_hbm, et_hbm,
             out_hbm, x_v, o_v, pe_v, comb_v, per_v, wf_v, bf_v, et_v, ev_v,
             xs0, xs1, ps0, ps1, os0, os1, evs, s_per, s_wf, s_bf, s_et):
    wid = lax.axis_index("s") * 2 + lax.axis_index("c")
    base = wid * CHUNK

    x_sems = (xs0, xs1)
    p_sems = (ps0, ps1)
    o_sems = (os0, os1)

    def start_in(t, k):
        s0 = base + t * SUB
        pltpu.make_async_copy(x_hbm.at[pl.ds(s0, SUB)], x_v.at[k], x_sems[k]).start()
        pltpu.make_async_copy(pe_hbm.at[pl.ds(s0, SUB)], pe_v.at[k], p_sems[k]).start()

    def wait_in(t, k):
        s0 = base + t * SUB
        pltpu.make_async_copy(x_hbm.at[pl.ds(s0, SUB)], x_v.at[k], x_sems[k]).wait()
        pltpu.make_async_copy(pe_hbm.at[pl.ds(s0, SUB)], pe_v.at[k], p_sems[k]).wait()

    def start_out(t, k):
        s0 = base + t * SUB
        pltpu.make_async_copy(o_v.at[k], out_hbm.at[pl.ds(s0, SUB)], o_sems[k]).start()

    def wait_out(t, k):
        s0 = base + t * SUB
        pltpu.make_async_copy(o_v.at[k], out_hbm.at[pl.ds(s0, SUB)], o_sems[k]).wait()

    start_in(0, 0)
    pltpu.make_async_copy(ev_hbm.at[pl.ds(base * B, CHUNK * B)], ev_v, evs).start()
    pltpu.make_async_copy(period_hbm, per_v.at[0, pl.ds(0, B)], s_per).start()
    pltpu.make_async_copy(wf_hbm, wf_v, s_wf).start()
    pltpu.make_async_copy(bf_hbm, bf_v, s_bf).start()
    pltpu.make_async_copy(et_hbm, et_v, s_et).start()
    pltpu.make_async_copy(period_hbm, per_v.at[0, pl.ds(0, B)], s_per).wait()
    pltpu.make_async_copy(wf_hbm, wf_v, s_wf).wait()
    pltpu.make_async_copy(bf_hbm, bf_v, s_bf).wait()
    pltpu.make_async_copy(et_hbm, et_v, s_et).wait()
    pltpu.make_async_copy(ev_hbm.at[pl.ds(base * B, CHUNK * B)], ev_v, evs).wait()

    for e in range(2):
        for b in range(B):
            p = jnp.take(per_v[0, :], jnp.full((L,), b, jnp.int32))
            def mk(j, _, e=e, b=b, p=p):
                sl = pl.ds(j * L, L)
                comb_v[e * B + b, sl] = (p * wf_v[0, sl] + bf_v[0, sl]
                                         + et_v[e, sl])
                return 0
            lax.fori_loop(0, D // L, mk, 0, unroll=4)

    def pair(tp, _):
        for phase in range(2):
            t = 2 * tp + phase
            k = phase
            @pl.when(t + 1 < T)
            def _():
                start_in(t + 1, 1 - k)
            wait_in(t, k)
            @pl.when(t >= 2)
            def _():
                wait_out(t - 2, k)
            vec = ev_v[pl.ds(t * SUB * B, L)]
            rs = [vec[lane] * B + (lane % B) for lane in range(L)]
            @plsc.parallel_loop(0, D // L, unroll=4)
            def dslice(j, rs=rs, k=k):
                sl = pl.ds(j * L, L)
                pes = [pe_v[k, s, sl] for s in range(SUB)]
                for lane in range(L):
                    s = lane // B
                    b = lane % B
                    o_v[k, s, b, sl] = (x_v[k, s, b, sl] + pes[s]
                                        + comb_v[rs[lane], sl])
            start_out(t, k)
        return 0

    lax.fori_loop(0, T // 2, pair, 0)
    wait_out(T - 2, 0)
    wait_out(T - 1, 1)


def kernel(x, period, events, W_freq, b_freq, event_table, pe):
    seq_len, batch, d = x.shape
    ev_flat = events.astype(jnp.int32).reshape(seq_len * batch)
    wf2 = W_freq.reshape(1, d)
    bf2 = b_freq.reshape(1, d)

    mesh = plsc.VectorSubcoreMesh(core_axis_name="c", subcore_axis_name="s")
    k = functools.partial(
        pl.kernel,
        mesh=mesh,
        out_type=jax.ShapeDtypeStruct((seq_len, batch, d), jnp.float32),
        scratch_types=[
            pltpu.VMEM((2, SUB, batch, d), jnp.float32),
            pltpu.VMEM((2, SUB, batch, d), jnp.float32),
            pltpu.VMEM((2, SUB, d), jnp.float32),
            pltpu.VMEM((2 * batch, d), jnp.float32),
            pltpu.VMEM((1, L), jnp.float32),
            pltpu.VMEM((1, d), jnp.float32),
            pltpu.VMEM((1, d), jnp.float32),
            pltpu.VMEM((2, d), jnp.float32),
            pltpu.VMEM((CHUNK * B,), jnp.int32),
            pltpu.SemaphoreType.DMA,
            pltpu.SemaphoreType.DMA,
            pltpu.SemaphoreType.DMA,
            pltpu.SemaphoreType.DMA,
            pltpu.SemaphoreType.DMA,
            pltpu.SemaphoreType.DMA,
            pltpu.SemaphoreType.DMA,
            pltpu.SemaphoreType.DMA,
            pltpu.SemaphoreType.DMA,
            pltpu.SemaphoreType.DMA,
            pltpu.SemaphoreType.DMA,
        ],
    )(_sc_body)
    return k(x, pe, ev_flat, period, wf2, bf2, event_table)
